# Initial kernel scaffold; baseline (speedup 1.0000x reference)
import jax
import jax.numpy as jnp
from jax.experimental import pallas as pl


def kernel(X, Feature, I_list, Node_is_leaf, enc_W1, enc_b1, enc_W2, enc_b2, enc_W3, enc_b3, dec_W1, dec_b1, dec_W2, dec_b2, dec_W3, dec_b3):
    raise NotImplementedError("write your pallas kernel here")


# speedup vs baseline: 3.9960x; 3.9960x over previous
import functools

import jax
import jax.numpy as jnp
from jax import lax
from jax.experimental import pallas as pl
from jax.experimental.pallas import tpu as pltpu
from jax.experimental.pallas import tpu_sc as plsc

_NF = 16
_N = 100000
_NLEVEL = 10
_NI = 65536
_NC = 2
_NS = 16
_NW = _NC * _NS
_BLK = 2048


def _sc_mesh():
    return plsc.VectorSubcoreMesh(core_axis_name="c", subcore_axis_name="s")


_SC_PARAMS = pltpu.CompilerParams(use_tc_tiling_on_sc=False)


def _wid():
    return lax.axis_index("s") * _NC + lax.axis_index("c")


def _global_barrier(bsem):
    plsc.subcore_barrier()
    pltpu.core_barrier(bsem, core_axis_name="c")
    plsc.subcore_barrier()


def _xgather_body(xp_hbm, it_hbm, out_hbm, idx_v, rows_v, sem):
    n = _NI // _NW
    base = _wid() * n
    for g in range(_NLEVEL * 3):
        lvl, col = divmod(g, 3)
        pltpu.sync_copy(it_hbm.at[lvl * 3 + col, 0, pl.ds(base, n)], idx_v)
        pltpu.async_copy(xp_hbm.at[idx_v], rows_v, sem).wait()
        pltpu.sync_copy(rows_v, out_hbm.at[g, pl.ds(base, n)])


def _fgather0_body(f_hbm, it_hbm, out_hbm, idx_v, rows_v, sem):
    n = _NI // _NW
    base = _wid() * n
    for col in range(2):
        pltpu.sync_copy(it_hbm.at[col, 0, pl.ds(base, n)], idx_v)
        pltpu.async_copy(f_hbm.at[idx_v], rows_v, sem).wait()
        pltpu.sync_copy(rows_v, out_hbm.at[col, pl.ds(base, n)])


def _make_scatter_body(lvl):
    def body(f_in, u, it_hbm, f_out, g_out, fbuf, idx_v, rows_v, sem, bsem):
        w = _wid()
        crows = _N // _NW
        cb = w * crows
        pltpu.sync_copy(f_in.at[pl.ds(cb, crows)], fbuf)
        pltpu.sync_copy(fbuf, f_out.at[pl.ds(cb, crows)])
        _global_barrier(bsem)
        n = _NI // _NW
        base = w * n
        for col in range(3):
            pltpu.sync_copy(it_hbm.at[lvl * 3 + col, 0, pl.ds(base, n)], idx_v)
            pltpu.sync_copy(u.at[col, pl.ds(base, n)], rows_v)
            pltpu.async_copy(rows_v, f_out.at[idx_v], sem).wait()
            _global_barrier(bsem)
        for col in range(2):
            pltpu.sync_copy(it_hbm.at[(lvl + 1) * 3 + col, 0, pl.ds(base, n)], idx_v)
            pltpu.async_copy(f_out.at[idx_v], rows_v, sem).wait()
            pltpu.sync_copy(rows_v, g_out.at[col, pl.ds(base, n)])
    return body


def _xgather(xp, it):
    n = _NI // _NW
    return pl.kernel(
        _xgather_body,
        compiler_params=_SC_PARAMS,
        out_type=jax.ShapeDtypeStruct((_NLEVEL * 3, _NI, 8), jnp.float32),
        mesh=_sc_mesh(),
        scratch_types=[
            pltpu.VMEM((n,), jnp.int32),
            pltpu.VMEM((n, 8), jnp.float32),
            pltpu.SemaphoreType.DMA,
        ],
    )(xp, it)


def _fgather0(f, it):
    n = _NI // _NW
    return pl.kernel(
        _fgather0_body,
        compiler_params=_SC_PARAMS,
        out_type=jax.ShapeDtypeStruct((2, _NI, _NF), jnp.float32),
        mesh=_sc_mesh(),
        scratch_types=[
            pltpu.VMEM((n,), jnp.int32),
            pltpu.VMEM((n, _NF), jnp.float32),
            pltpu.SemaphoreType.DMA,
        ],
    )(f, it)


def _scatter_gather(lvl, f_cur, u, it):
    n = _NI // _NW
    return pl.kernel(
        _make_scatter_body(lvl),
        compiler_params=_SC_PARAMS,
        out_type=[
            jax.ShapeDtypeStruct((_N, _NF), jnp.float32),
            jax.ShapeDtypeStruct((2, _NI, _NF), jnp.float32),
        ],
        mesh=_sc_mesh(),
        scratch_types=[
            pltpu.VMEM((_N // _NW, _NF), jnp.float32),
            pltpu.VMEM((n,), jnp.int32),
            pltpu.VMEM((n, _NF), jnp.float32),
            pltpu.SemaphoreType.DMA,
            pltpu.SemaphoreType.REGULAR,
        ],
    )(f_cur, u, it)


def _mlp_level_body(p0_ref, p1_ref, p2_ref, f0_ref, f1_ref,
                    w1p_ref, w1f_ref, w2_ref, w3_ref, b1_ref, b2_ref, b3_ref,
                    v1f_ref, v1p_ref, v2_ref, v3_ref, c1_ref, c2_ref, c3_ref,
                    u_ref, loss_ref):
    blk = pl.program_id(0)
    p0 = p0_ref[0]
    p1 = p1_ref[0]
    p2 = p2_ref[0]
    f0 = f0_ref[0]
    f1 = f1_ref[0]
    dot = functools.partial(jnp.dot, preferred_element_type=jnp.float32)

    def enc(p, f):
        h = jax.nn.relu(dot(p, w1p_ref[...]) + dot(f, w1f_ref[...]) + b1_ref[...])
        h = jax.nn.relu(dot(h, w2_ref[...]) + b2_ref[...])
        return dot(h, w3_ref[...]) + b3_ref[...]

    father_f = enc(p0, f0) + enc(p1, f1)
    d = jax.nn.relu(dot(father_f, v1f_ref[...]) + dot(p2, v1p_ref[...]) + c1_ref[...])
    d = jax.nn.relu(dot(d, v2_ref[...]) + c2_ref[...])
    out = dot(d, v3_ref[...]) + c3_ref[...]

    nf = _NF
    u_ref[0] = out[:, :nf]
    u_ref[1] = out[:, nf + 6:2 * nf + 6]
    u_ref[2] = father_f

    def side_loss(p, o0):
        pred_xywh = jnp.concatenate(
            [jnp.tanh(out[:, o0:o0 + 2]), jax.nn.sigmoid(out[:, o0 + 2:o0 + 4])],
            axis=1)
        e_xywh = jnp.sum((p[:, 0:4] - pred_xywh) ** 2) * 0.5
        e_a = jnp.sum((p[:, 4:5] - out[:, o0 + 4:o0 + 5]) ** 2)
        return e_xywh + e_a

    part = side_loss(p0, nf) + side_loss(p1, 2 * nf + 6)

    @pl.when(blk == 0)
    def _():
        loss_ref[...] = jnp.zeros((1, 1), jnp.float32)

    loss_ref[...] += jnp.full((1, 1), part, jnp.float32)


def _mlp_level(lvl, xg, g, wts):
    nb = _NI // _BLK
    xspec = lambda c: pl.BlockSpec((1, _BLK, 8), lambda b, c=c: (3 * lvl + c, b, 0))
    gspec = lambda c: pl.BlockSpec((1, _BLK, _NF), lambda b, c=c: (c, b, 0))
    wspec = lambda a: pl.BlockSpec(a.shape, lambda b: (0,) * a.ndim)
    in_specs = [xspec(0), xspec(1), xspec(2), gspec(0), gspec(1)]
    in_specs += [wspec(w) for w in wts]
    return pl.pallas_call(
        _mlp_level_body,
        grid=(nb,),
        in_specs=in_specs,
        out_specs=[
            pl.BlockSpec((3, _BLK, _NF), lambda b: (0, b, 0)),
            pl.BlockSpec((1, 1), lambda b: (0, 0)),
        ],
        out_shape=[
            jax.ShapeDtypeStruct((3, _NI, _NF), jnp.float32),
            jax.ShapeDtypeStruct((1, 1), jnp.float32),
        ],
    )(xg, xg, xg, g, g, *wts)


def kernel(X, Feature, I_list, Node_is_leaf,
           enc_W1, enc_b1, enc_W2, enc_b2, enc_W3, enc_b3,
           dec_W1, dec_b1, dec_W2, dec_b2, dec_W3, dec_b3):
    xp = jnp.pad(X[0], ((0, 0), (0, 3)))
    f0 = Feature[0]
    it = jnp.transpose(I_list[:, 0], (0, 2, 1)).reshape(_NLEVEL * 3, 1, _NI)

    pad8 = lambda w: jnp.pad(w, ((0, 8 - w.shape[0]), (0, 0)))
    wts = (
        pad8(enc_W1[:5]), enc_W1[5:], enc_W2, enc_W3,
        enc_b1.reshape(1, -1), enc_b2.reshape(1, -1), enc_b3.reshape(1, -1),
        dec_W1[:_NF], pad8(dec_W1[_NF:]), dec_W2, dec_W3,
        dec_b1.reshape(1, -1), dec_b2.reshape(1, -1), dec_b3.reshape(1, -1),
    )

    xg = _xgather(xp, it)
    g = _fgather0(f0, it)

    f_cur = f0
    total = jnp.float32(0.0)
    for lvl in range(_NLEVEL):
        u, part = _mlp_level(lvl, xg, g, wts)
        total = total + part[0, 0]
        if lvl < _NLEVEL - 1:
            f_cur, g = _scatter_gather(lvl, f_cur, u, it)

    loss_p = total / jnp.float32(_NI * _NLEVEL)
    zero = jnp.float32(0.0)
    return (loss_p, zero, loss_p, zero)


# speedup vs baseline: 4.7754x; 1.1951x over previous
import functools

import jax
import jax.numpy as jnp
from jax import lax
from jax.experimental import pallas as pl
from jax.experimental.pallas import tpu as pltpu
from jax.experimental.pallas import tpu_sc as plsc
from jax._src.pallas import mpmd as _mpmd

_NF = 16
_N = 100000
_NLEVEL = 10
_NI = 65536
_NC = 2
_NS = 16
_NW = _NC * _NS
_BLK = 4096


def _sc_mesh():
    return plsc.VectorSubcoreMesh(core_axis_name="c", subcore_axis_name="s")


_SC_PARAMS = pltpu.CompilerParams(use_tc_tiling_on_sc=False)


def _wid():
    return lax.axis_index("s") * _NC + lax.axis_index("c")


def _global_barrier(bsem):
    plsc.subcore_barrier()
    pltpu.core_barrier(bsem, core_axis_name="c")
    plsc.subcore_barrier()


def _gather0_body(xp_hbm, f_hbm, it_hbm, xg_hbm, g_hbm, idx_v, xrows_v,
                  frows_v, sem):
    n = _NI // _NW
    base = _wid() * n
    for col in range(3):
        pltpu.sync_copy(it_hbm.at[col, 0, pl.ds(base, n)], idx_v)
        pltpu.async_copy(xp_hbm.at[idx_v], xrows_v, sem).wait()
        pltpu.sync_copy(xrows_v, xg_hbm.at[col, pl.ds(base, n)])
    for col in range(2):
        pltpu.sync_copy(it_hbm.at[col, 0, pl.ds(base, n)], idx_v)
        pltpu.async_copy(f_hbm.at[idx_v], frows_v, sem).wait()
        pltpu.sync_copy(frows_v, g_hbm.at[col, pl.ds(base, n)])


def _xgather_rest_body(xp_hbm, it_hbm, out_hbm, idx_v, rows_v, sem):
    n = _NI // _NW
    base = _wid() * n
    for g in range(3, _NLEVEL * 3):
        pltpu.sync_copy(it_hbm.at[g, 0, pl.ds(base, n)], idx_v)
        pltpu.async_copy(xp_hbm.at[idx_v], rows_v, sem).wait()
        pltpu.sync_copy(rows_v, out_hbm.at[g - 3, pl.ds(base, n)])


def _make_scatter_body(lvl):
    def body(f_in, u, it_hbm, f_out, g_out, idx_v, rows_v, sem, bsem):
        del f_in
        n = _NI // _NW
        base = _wid() * n
        for col in range(3):
            pltpu.sync_copy(it_hbm.at[lvl * 3 + col, 0, pl.ds(base, n)], idx_v)
            pltpu.sync_copy(u.at[col, pl.ds(base, n)], rows_v)
            pltpu.async_copy(rows_v, f_out.at[idx_v], sem).wait()
            _global_barrier(bsem)
        for col in range(2):
            pltpu.sync_copy(it_hbm.at[(lvl + 1) * 3 + col, 0, pl.ds(base, n)], idx_v)
            pltpu.async_copy(f_out.at[idx_v], rows_v, sem).wait()
            pltpu.sync_copy(rows_v, g_out.at[col, pl.ds(base, n)])
    return body


def _gather0(xp, f0, it):
    n = _NI // _NW
    return pl.kernel(
        _gather0_body,
        out_type=[
            jax.ShapeDtypeStruct((3, _NI, 8), jnp.float32),
            jax.ShapeDtypeStruct((2, _NI, _NF), jnp.float32),
        ],
        mesh=_sc_mesh(),
        compiler_params=_SC_PARAMS,
        scratch_types=[
            pltpu.VMEM((n,), jnp.int32),
            pltpu.VMEM((n, 8), jnp.float32),
            pltpu.VMEM((n, _NF), jnp.float32),
            pltpu.SemaphoreType.DMA,
        ],
    )(xp, f0, it)


def _xgather_rest(xp, it):
    n = _NI // _NW
    return pl.kernel(
        _xgather_rest_body,
        out_type=jax.ShapeDtypeStruct(((_NLEVEL - 1) * 3, _NI, 8), jnp.float32),
        mesh=_sc_mesh(),
        compiler_params=_SC_PARAMS,
        scratch_types=[
            pltpu.VMEM((n,), jnp.int32),
            pltpu.VMEM((n, 8), jnp.float32),
            pltpu.SemaphoreType.DMA,
        ],
    )(xp, it)


def _scatter_gather(lvl, f_cur, u, it):
    n = _NI // _NW
    return _mpmd._mpmd_map(
        [(_sc_mesh(), _make_scatter_body(lvl))],
        [
            jax.ShapeDtypeStruct((_N, _NF), jnp.float32),
            jax.ShapeDtypeStruct((2, _NI, _NF), jnp.float32),
        ],
        input_output_aliases={0: 0},
        compiler_params=_SC_PARAMS,
        scratch_types=[
            pltpu.VMEM((n,), jnp.int32),
            pltpu.VMEM((n, _NF), jnp.float32),
            pltpu.SemaphoreType.DMA,
            pltpu.SemaphoreType.REGULAR,
        ],
    )(f_cur, u, it)


def _mlp_level_body(p0_ref, p1_ref, p2_ref, f0_ref, f1_ref,
                    w1p_ref, w1f_ref, w2_ref, w3_ref, b1_ref, b2_ref, b3_ref,
                    v1f_ref, v1p_ref, v2_ref, v3_ref, c1_ref, c2_ref, c3_ref,
                    u_ref, loss_ref):
    blk = pl.program_id(0)
    p0 = p0_ref[0]
    p1 = p1_ref[0]
    p2 = p2_ref[0]
    f0 = f0_ref[0]
    f1 = f1_ref[0]
    dot = functools.partial(jnp.dot, preferred_element_type=jnp.float32)

    def enc(p, f):
        h = jax.nn.relu(dot(p, w1p_ref[...]) + dot(f, w1f_ref[...]) + b1_ref[...])
        h = jax.nn.relu(dot(h, w2_ref[...]) + b2_ref[...])
        return dot(h, w3_ref[...]) + b3_ref[...]

    father_f = enc(p0, f0) + enc(p1, f1)
    d = jax.nn.relu(dot(father_f, v1f_ref[...]) + dot(p2, v1p_ref[...]) + c1_ref[...])
    d = jax.nn.relu(dot(d, v2_ref[...]) + c2_ref[...])
    out = dot(d, v3_ref[...]) + c3_ref[...]

    u_ref[0] = out[:, 0:_NF]
    u_ref[1] = out[:, _NF:2 * _NF]
    u_ref[2] = father_f

    pr = out[:, 2 * _NF:2 * _NF + 10]
    t = jnp.tanh(pr)
    s = jax.nn.sigmoid(pr)
    lane = jax.lax.broadcasted_iota(jnp.int32, (1, 10), 1) % 5
    pred = jnp.where(lane < 2, t, jnp.where(lane < 4, s, pr))
    wgt = jnp.where(lane == 4, 1.0, 0.5)
    q = jnp.concatenate([p0[:, 0:5], p1[:, 0:5]], axis=1)
    part = jnp.sum(((q - pred) ** 2) * wgt)

    @pl.when(blk == 0)
    def _():
        loss_ref[...] = jnp.zeros((1, 1), jnp.float32)

    loss_ref[...] += jnp.full((1, 1), part, jnp.float32)


def _mlp_level(lvl, xg, g, wts):
    nb = _NI // _BLK
    xoff = 0 if lvl == 0 else 3 * (lvl - 1)
    xspec = lambda c: pl.BlockSpec((1, _BLK, 8), lambda b, c=c: (xoff + c, b, 0))
    gspec = lambda c: pl.BlockSpec((1, _BLK, _NF), lambda b, c=c: (c, b, 0))
    wspec = lambda a: pl.BlockSpec(a.shape, lambda b: (0,) * a.ndim)
    in_specs = [xspec(0), xspec(1), xspec(2), gspec(0), gspec(1)]
    in_specs += [wspec(w) for w in wts]
    return pl.pallas_call(
        _mlp_level_body,
        grid=(nb,),
        in_specs=in_specs,
        out_specs=[
            pl.BlockSpec((3, _BLK, _NF), lambda b: (0, b, 0)),
            pl.BlockSpec((1, 1), lambda b: (0, 0)),
        ],
        out_shape=[
            jax.ShapeDtypeStruct((3, _NI, _NF), jnp.float32),
            jax.ShapeDtypeStruct((1, 1), jnp.float32),
        ],
    )(xg, xg, xg, g, g, *wts)


def kernel(X, Feature, I_list, Node_is_leaf,
           enc_W1, enc_b1, enc_W2, enc_b2, enc_W3, enc_b3,
           dec_W1, dec_b1, dec_W2, dec_b2, dec_W3, dec_b3):
    xp = jnp.pad(X[0], ((0, 0), (0, 3)))
    f0 = Feature[0]
    it = jnp.transpose(I_list[:, 0], (0, 2, 1)).reshape(_NLEVEL * 3, 1, _NI)

    perm = list(range(16)) + list(range(22, 38)) + [16, 17, 18, 19, 20,
                                                    38, 39, 40, 41, 42]
    dec_W3p = dec_W3[:, jnp.array(perm)]
    dec_b3p = dec_b3[jnp.array(perm)]

    pad8 = lambda w: jnp.pad(w, ((0, 8 - w.shape[0]), (0, 0)))
    wts = (
        pad8(enc_W1[:5]), enc_W1[5:], enc_W2, enc_W3,
        enc_b1.reshape(1, -1), enc_b2.reshape(1, -1), enc_b3.reshape(1, -1),
        dec_W1[:_NF], pad8(dec_W1[_NF:]), dec_W2, dec_W3p,
        dec_b1.reshape(1, -1), dec_b2.reshape(1, -1), dec_b3p.reshape(1, -1),
    )

    xg0, g = _gather0(xp, f0, it)
    xgr = _xgather_rest(xp, it)

    f_cur = f0
    total = jnp.float32(0.0)
    for lvl in range(_NLEVEL):
        u, part = _mlp_level(lvl, xg0 if lvl == 0 else xgr, g, wts)
        total = total + part[0, 0]
        if lvl < _NLEVEL - 1:
            f_cur, g = _scatter_gather(lvl, f_cur, u, it)

    loss_p = total / jnp.float32(_NI * _NLEVEL)
    zero = jnp.float32(0.0)
    return (loss_p, zero, loss_p, zero)


# speedup vs baseline: 5.4251x; 1.1361x over previous
import functools

import jax
import jax.numpy as jnp
from jax import lax
from jax.experimental import pallas as pl
from jax.experimental.pallas import tpu as pltpu
from jax.experimental.pallas import tpu_sc as plsc
from jax._src.pallas import mpmd as _mpmd

_NF = 16
_N = 100000
_NLEVEL = 10
_NI = 65536
_NC = 2
_NS = 16
_NW = _NC * _NS
_BLK = 8192


def _sc_mesh():
    return plsc.VectorSubcoreMesh(core_axis_name="c", subcore_axis_name="s")


_SC_PARAMS = pltpu.CompilerParams(use_tc_tiling_on_sc=False)


def _wid():
    return lax.axis_index("s") * _NC + lax.axis_index("c")


def _global_barrier(bsem):
    plsc.subcore_barrier()
    pltpu.core_barrier(bsem, core_axis_name="c")
    plsc.subcore_barrier()


def _gather0_body(xp_hbm, f_hbm, it_hbm, xg_hbm, g_hbm, idx_v, xrows_v,
                  frows_v, sem):
    n = _NI // _NW
    base = _wid() * n
    for col in range(3):
        pltpu.sync_copy(it_hbm.at[col, 0, pl.ds(base, n)], idx_v)
        pltpu.async_copy(xp_hbm.at[idx_v], xrows_v, sem).wait()
        pltpu.sync_copy(xrows_v, xg_hbm.at[col, pl.ds(base, n)])
    for col in range(2):
        pltpu.sync_copy(it_hbm.at[col, 0, pl.ds(base, n)], idx_v)
        pltpu.async_copy(f_hbm.at[idx_v], frows_v, sem).wait()
        pltpu.sync_copy(frows_v, g_hbm.at[col, pl.ds(base, n)])


def _xgather_rest_body(xp_hbm, it_hbm, out_hbm, idx_v, rows_v, sem):
    n = _NI // _NW
    base = _wid() * n
    for g in range(3, _NLEVEL * 3):
        pltpu.sync_copy(it_hbm.at[g, 0, pl.ds(base, n)], idx_v)
        pltpu.async_copy(xp_hbm.at[idx_v], rows_v, sem).wait()
        pltpu.sync_copy(rows_v, out_hbm.at[g - 3, pl.ds(base, n)])


def _make_scatter_body(lvl):
    def body(f_in, u, it_hbm, f_out, g_out, idx_v, rows_v, sem, bsem):
        del f_in
        n = _NI // _NW
        base = _wid() * n
        for col in range(3):
            pltpu.sync_copy(it_hbm.at[lvl * 3 + col, 0, pl.ds(base, n)], idx_v)
            pltpu.sync_copy(u.at[col, pl.ds(base, n)], rows_v)
            pltpu.async_copy(rows_v, f_out.at[idx_v], sem).wait()
            _global_barrier(bsem)
        for col in range(2):
            pltpu.sync_copy(it_hbm.at[(lvl + 1) * 3 + col, 0, pl.ds(base, n)], idx_v)
            pltpu.async_copy(f_out.at[idx_v], rows_v, sem).wait()
            pltpu.sync_copy(rows_v, g_out.at[col, pl.ds(base, n)])
    return body


def _gather0(xp, f0, it):
    n = _NI // _NW
    return pl.kernel(
        _gather0_body,
        out_type=[
            jax.ShapeDtypeStruct((3, _NI, 8), jnp.float32),
            jax.ShapeDtypeStruct((2, _NI, _NF), jnp.float32),
        ],
        mesh=_sc_mesh(),
        compiler_params=_SC_PARAMS,
        scratch_types=[
            pltpu.VMEM((n,), jnp.int32),
            pltpu.VMEM((n, 8), jnp.float32),
            pltpu.VMEM((n, _NF), jnp.float32),
            pltpu.SemaphoreType.DMA,
        ],
    )(xp, f0, it)


def _xgather_rest(xp, it):
    n = _NI // _NW
    return pl.kernel(
        _xgather_rest_body,
        out_type=jax.ShapeDtypeStruct(((_NLEVEL - 1) * 3, _NI, 8), jnp.float32),
        mesh=_sc_mesh(),
        compiler_params=_SC_PARAMS,
        scratch_types=[
            pltpu.VMEM((n,), jnp.int32),
            pltpu.VMEM((n, 8), jnp.float32),
            pltpu.SemaphoreType.DMA,
        ],
    )(xp, it)


def _scatter_gather(lvl, f_cur, u, it):
    n = _NI // _NW
    return _mpmd._mpmd_map(
        [(_sc_mesh(), _make_scatter_body(lvl))],
        [
            jax.ShapeDtypeStruct((_N, _NF), jnp.float32),
            jax.ShapeDtypeStruct((2, _NI, _NF), jnp.float32),
        ],
        input_output_aliases={0: 0},
        compiler_params=_SC_PARAMS,
        scratch_types=[
            pltpu.VMEM((n,), jnp.int32),
            pltpu.VMEM((n, _NF), jnp.float32),
            pltpu.SemaphoreType.DMA,
            pltpu.SemaphoreType.REGULAR,
        ],
    )(f_cur, u, it)


def _mlp_level_body(p0_ref, p1_ref, p2_ref, f0_ref, f1_ref,
                    w1p_ref, w1f_ref, w2_ref, w3_ref, b1_ref, b2_ref, b3_ref,
                    v1f_ref, v1p_ref, v2_ref, v3l_ref, v3r_ref, v3p_ref,
                    q0_ref, q1_ref, c1_ref, c2_ref, c3l_ref, c3r_ref, c3p_ref,
                    u_ref, loss_ref):
    blk = pl.program_id(0)
    p0 = p0_ref[0]
    p1 = p1_ref[0]
    p2 = p2_ref[0]
    f0 = f0_ref[0]
    f1 = f1_ref[0]
    dot = functools.partial(jnp.dot, preferred_element_type=jnp.float32)

    def enc(p, f):
        h = jax.nn.relu(dot(p, w1p_ref[...]) + dot(f, w1f_ref[...]) + b1_ref[...])
        h = jax.nn.relu(dot(h, w2_ref[...]) + b2_ref[...])
        return dot(h, w3_ref[...]) + b3_ref[...]

    father_f = enc(p0, f0) + enc(p1, f1)
    d = jax.nn.relu(dot(father_f, v1f_ref[...]) + dot(p2, v1p_ref[...]) + c1_ref[...])
    d = jax.nn.relu(dot(d, v2_ref[...]) + c2_ref[...])

    u_ref[0] = dot(d, v3l_ref[...]) + c3l_ref[...]
    u_ref[1] = dot(d, v3r_ref[...]) + c3r_ref[...]
    u_ref[2] = father_f

    pr = dot(d, v3p_ref[...]) + c3p_ref[...]
    q = dot(p0, q0_ref[...]) + dot(p1, q1_ref[...])
    k = jax.lax.broadcasted_iota(jnp.int32, (1, 16 * _NF), 1) % 16
    m = jnp.where(k < 5, k, k - 5)
    live = k < 10
    pred = jnp.where((m < 2) & live, jnp.tanh(pr),
                     jnp.where((m >= 2) & (m < 4) & live, jax.nn.sigmoid(pr), pr))
    wgt = jnp.where(live, jnp.where(m == 4, 1.0, 0.5), 0.0)
    part = jnp.sum(((q - pred) ** 2) * wgt)

    @pl.when(blk == 0)
    def _():
        loss_ref[...] = jnp.zeros((1, 1), jnp.float32)

    loss_ref[...] += jnp.full((1, 1), part, jnp.float32)


def _mlp_level(lvl, xg, g, wts):
    nb = _NI // _BLK
    r = _BLK // 16
    xoff = 0 if lvl == 0 else 3 * (lvl - 1)
    xspec = lambda c: pl.BlockSpec((1, r, 128), lambda b, c=c: (xoff + c, b, 0))
    gspec = lambda c: pl.BlockSpec((1, r, 16 * _NF), lambda b, c=c: (c, b, 0))
    wspec = lambda a: pl.BlockSpec(a.shape, lambda b: (0,) * a.ndim)
    in_specs = [xspec(0), xspec(1), xspec(2), gspec(0), gspec(1)]
    in_specs += [wspec(w) for w in wts]
    return pl.pallas_call(
        _mlp_level_body,
        grid=(nb,),
        in_specs=in_specs,
        out_specs=[
            pl.BlockSpec((3, r, 16 * _NF), lambda b: (0, b, 0)),
            pl.BlockSpec((1, 1), lambda b: (0, 0)),
        ],
        out_shape=[
            jax.ShapeDtypeStruct((3, _NI // 16, 16 * _NF), jnp.float32),
            jax.ShapeDtypeStruct((1, 1), jnp.float32),
        ],
    )(xg, xg, xg, g, g, *wts)


def kernel(X, Feature, I_list, Node_is_leaf,
           enc_W1, enc_b1, enc_W2, enc_b2, enc_W3, enc_b3,
           dec_W1, dec_b1, dec_W2, dec_b2, dec_W3, dec_b3):
    xp = jnp.pad(X[0], ((0, 0), (0, 3)))
    f0 = Feature[0]
    it = jnp.transpose(I_list[:, 0], (0, 2, 1)).reshape(_NLEVEL * 3, 1, _NI)

    eye16 = jnp.eye(16, dtype=jnp.float32)
    kron = lambda w: jnp.kron(eye16, w)
    tile = lambda b: jnp.tile(b, 16).reshape(1, -1)
    pad8 = lambda w: jnp.pad(w, ((0, 8 - w.shape[0]), (0, 0)))
    perm10 = jnp.array([16, 17, 18, 19, 20, 38, 39, 40, 41, 42])
    v3p = jnp.pad(dec_W3[:, perm10], ((0, 0), (0, 6)))
    c3p = jnp.pad(dec_b3[perm10], (0, 6))
    eye5 = jnp.eye(5, dtype=jnp.float32)
    q0 = jnp.zeros((8, 16), jnp.float32).at[:5, :5].set(eye5)
    q1 = jnp.zeros((8, 16), jnp.float32).at[:5, 5:10].set(eye5)

    wts = (
        kron(pad8(enc_W1[:5])), kron(enc_W1[5:]), kron(enc_W2), kron(enc_W3),
        tile(enc_b1), tile(enc_b2), tile(enc_b3),
        kron(dec_W1[:_NF]), kron(pad8(dec_W1[_NF:])), kron(dec_W2),
        kron(dec_W3[:, 0:_NF]), kron(dec_W3[:, 22:22 + _NF]), kron(v3p),
        kron(q0), kron(q1),
        tile(dec_b1), tile(dec_b2),
        tile(dec_b3[0:_NF]), tile(dec_b3[22:22 + _NF]), tile(c3p),
    )

    xg0, g = _gather0(xp, f0, it)
    xgr = _xgather_rest(xp, it)
    xg0 = xg0.reshape(3, _NI // 16, 128)
    xgr = xgr.reshape((_NLEVEL - 1) * 3, _NI // 16, 128)

    f_cur = f0
    total = jnp.float32(0.0)
    for lvl in range(_NLEVEL):
        gp = g.reshape(2, _NI // 16, 16 * _NF)
        u, part = _mlp_level(lvl, xg0 if lvl == 0 else xgr, gp, wts)
        total = total + part[0, 0]
        if lvl < _NLEVEL - 1:
            f_cur, g = _scatter_gather(lvl, f_cur, u.reshape(3, _NI, _NF), it)

    loss_p = total / jnp.float32(_NI * _NLEVEL)
    zero = jnp.float32(0.0)
    return (loss_p, zero, loss_p, zero)
